# baseline (speedup 1.0000x reference)
import jax
import jax.numpy as jnp
from jax.experimental import pallas as pl
from jax.experimental.pallas import tpu as pltpu

B = 4096
F = 16384
BR = 128


def _body(feat_ref, w_ref, out_ref, acc_ref):
    i = pl.program_id(0)
    part = jnp.sum(feat_ref[...] * w_ref[...], axis=1)
    acc_ref[0, pl.ds(i * BR, BR)] = part

    @pl.when(i == pl.num_programs(0) - 1)
    def _():
        s = acc_ref[...]
        m = jnp.max(s)
        e = jnp.exp(s - m)
        out_ref[...] = e / jnp.sum(e)


def kernel(feature_vector, linear):
    w_row = linear.reshape(1, F)
    probs = pl.pallas_call(
        _body,
        grid=(B // BR,),
        in_specs=[
            pl.BlockSpec((BR, F), lambda i: (i, 0)),
            pl.BlockSpec((1, F), lambda i: (0, 0)),
        ],
        out_specs=pl.BlockSpec((1, B), lambda i: (0, 0)),
        out_shape=jax.ShapeDtypeStruct((1, B), jnp.float32),
        scratch_shapes=[pltpu.VMEM((1, B), jnp.float32)],
    )(feature_vector, w_row)
    return probs.reshape(1, B, 1)
